# baseline (speedup 1.0000x reference)
import functools

import jax
import jax.numpy as jnp
from jax import lax
from jax.experimental import pallas as pl
from jax.experimental.pallas import tpu as pltpu
from jax.experimental.pallas import tpu_sc as plsc

_N = 10000
_E = 320000
_F_IN = 128
_HID = 16

_NC = 2
_NS = 16
_NW = _NC * _NS
_EPW = _E // _NW
_NCH = 125
_CH = 80
_NPAD = 10112
_STRIPE = _NPAD // _NS
_PK = _NPAD // 8
_PKN = _N // 8


def _sc_mesh():
    return plsc.VectorSubcoreMesh(
        core_axis_name="c", subcore_axis_name="s",
        num_cores=_NC, num_subcores=_NS)


def _sc_degree(e4):
    @functools.partial(
        pl.kernel,
        out_type=jax.ShapeDtypeStruct((_NW, _NPAD), jnp.float32),
        mesh=_sc_mesh(),
        scratch_types=[
            pltpu.VMEM((_NCH, _CH), jnp.int32),
            pltpu.VMEM((_NPAD,), jnp.float32),
        ],
        compiler_params=pltpu.CompilerParams(
            needs_layout_passes=False, use_tc_tiling_on_sc=False),
    )
    def deg_kernel(e_hbm, deg_hbm, dst_v, deg_v):
        cid = lax.axis_index("c")
        sid = lax.axis_index("s")
        wid = sid * _NC + cid
        pltpu.sync_copy(e_hbm.at[1, wid], dst_v)

        zero16 = jnp.zeros((16,), jnp.float32)

        @pl.loop(0, _NPAD // 16)
        def _zero(i):
            deg_v[pl.ds(i * 16, 16)] = zero16

        one16 = jnp.ones((16,), jnp.float32)

        @pl.loop(0, _NCH)
        def _hist(j):
            for k in range(_CH // 16):
                idx = dst_v[j, pl.ds(k * 16, 16)]
                plsc.addupdate_scatter(deg_v, [idx], one16)

        pltpu.sync_copy(deg_v, deg_hbm.at[wid])

    return deg_kernel(e4)


def _sc_aggregate(y, e4, zeros_pad):
    nbuf = 5
    pre = 3

    @functools.partial(
        pl.kernel,
        out_type=jax.ShapeDtypeStruct((_NC, _NPAD, _HID), jnp.float32),
        mesh=_sc_mesh(),
        scratch_types=[
            pltpu.VMEM((_NCH, _CH), jnp.int32),
            pltpu.VMEM((_NCH, _CH), jnp.int32),
            [pltpu.VMEM((_CH, _HID), jnp.float32) for _ in range(nbuf)],
            pltpu.VMEM((_STRIPE, _HID), jnp.float32),
            pltpu.VMEM_SHARED((_NPAD, _HID), jnp.float32),
            [pltpu.SemaphoreType.DMA for _ in range(nbuf)],
            [pltpu.SemaphoreType.DMA for _ in range(nbuf)],
        ],
        compiler_params=pltpu.CompilerParams(use_tc_tiling_on_sc=False),
    )
    def agg_kernel(y_hbm, e_hbm, zero_hbm, out_hbm,
                   src_v, dst_v, rows_v, stage_v, acc, gsem, ssem):
        cid = lax.axis_index("c")
        sid = lax.axis_index("s")
        wid = sid * _NC + cid

        pltpu.sync_copy(e_hbm.at[0, wid], src_v)
        pltpu.sync_copy(e_hbm.at[1, wid], dst_v)
        pltpu.sync_copy(zero_hbm.at[pl.ds(sid * _STRIPE, _STRIPE)],
                        acc.at[pl.ds(sid * _STRIPE, _STRIPE)])
        plsc.subcore_barrier()

        for b in range(pre):
            pltpu.async_copy(y_hbm.at[src_v.at[b]], rows_v[b], gsem[b])

        @pl.loop(0, _NCH // nbuf)
        def _blk(jj):
            for b in range(nbuf):
                c = jj * nbuf + b
                pltpu.make_async_copy(y_hbm.at[src_v.at[c]],
                                      rows_v[b], gsem[b]).wait()
                pltpu.async_copy(rows_v[b], acc.at[dst_v.at[c]],
                                 ssem[b], add=True)
                b2 = (b + pre) % nbuf

                @pl.when(c >= nbuf - pre)
                def _drain():
                    pltpu.make_async_copy(rows_v[b2], acc.at[dst_v.at[c]],
                                          ssem[b2]).wait()

                @pl.when(c < _NCH - pre)
                def _prefetch():
                    pltpu.async_copy(y_hbm.at[src_v.at[c + pre]],
                                     rows_v[b2], gsem[b2])

        for k in range(nbuf - pre):
            b2 = (_NCH + pre - nbuf + k) % nbuf
            pltpu.make_async_copy(rows_v[b2], acc.at[dst_v.at[0]],
                                  ssem[b2]).wait()

        plsc.subcore_barrier()
        pltpu.sync_copy(acc.at[pl.ds(sid * _STRIPE, _STRIPE)], stage_v)
        pltpu.sync_copy(stage_v,
                        out_hbm.at[cid, pl.ds(sid * _STRIPE, _STRIPE)])

    return agg_kernel(y, e4, zeros_pad)


def _tc_matmul1(xp, Wb1):
    def body(x_ref, w_ref, xw_ref):
        xw = jnp.dot(x_ref[...], w_ref[...], preferred_element_type=jnp.float32)
        xw_ref[0:_PKN] = xw
        xw_ref[_PKN:_PK] = jnp.zeros((_PK - _PKN, 128), jnp.float32)

    return pl.pallas_call(
        body,
        out_shape=jax.ShapeDtypeStruct((_PK, 128), jnp.float32),
    )(xp, Wb1)


def _tc_dinv(deg_part):
    def body(deg_ref, dinv_ref):
        deg = jnp.sum(deg_ref[...], axis=0, keepdims=True)
        dinv_ref[...] = lax.rsqrt(deg + 1.0)

    return pl.pallas_call(
        body,
        out_shape=jax.ShapeDtypeStruct((1, _NPAD), jnp.float32),
    )(deg_part)


def _tc_layer2(p, y1p, dinvp, b1p, Wb2):
    def body(p_ref, y_ref, d_ref, b_ref, w_ref, out_ref):
        d = d_ref[...]
        h = jnp.maximum(d * (p_ref[0] + p_ref[1] + y_ref[...]) + b_ref[...],
                        0.0)
        out_ref[...] = jnp.dot(
            h, w_ref[...], preferred_element_type=jnp.float32) * d

    return pl.pallas_call(
        body,
        out_shape=jax.ShapeDtypeStruct((_PK, 128), jnp.float32),
    )(p, y1p, dinvp, b1p, Wb2)


def _tc_layer3(p, y2p, dinvp, b2p, R):
    def body(p_ref, y_ref, d_ref, b_ref, r_ref, out_ref):
        o = d_ref[...] * (p_ref[0] + p_ref[1] + y_ref[...]) + b_ref[...]
        s = o - jnp.max(o)
        e = jnp.exp(s)
        r = r_ref[...]
        t = lax.dot_general(e, r, (((1,), (1,)), ((), ())),
                            preferred_element_type=jnp.float32)
        lsp = s - jnp.dot(jnp.log(t), r, preferred_element_type=jnp.float32)
        out_ref[...] = lsp[0:_PKN]

    return pl.pallas_call(
        body,
        out_shape=jax.ShapeDtypeStruct((_PKN, 128), jnp.float32),
    )(p, y2p, dinvp, b2p, R)


def kernel(x, edge_index, W1, b1, W2, b2):
    eye8 = jnp.eye(8, dtype=jnp.float32)
    Wb1 = jnp.kron(eye8, W1)
    Wb2 = jnp.kron(eye8, W2)
    R = jnp.kron(eye8, jnp.ones((1, 16), jnp.float32))
    b1p = jnp.tile(b1, 8).reshape(1, 128)
    b2p = jnp.tile(b2, 8).reshape(1, 128)
    xp = x.reshape(_PKN, 8 * _F_IN)
    e4 = edge_index.reshape(2, _NW, _NCH, _CH)
    zeros_pad = jnp.zeros((_NPAD, _HID), jnp.float32)

    deg_part = _sc_degree(e4)
    xwp = _tc_matmul1(xp, Wb1)
    dinv = _tc_dinv(deg_part)
    dinvp = jnp.broadcast_to(
        dinv.reshape(_PK, 8, 1), (_PK, 8, _HID)).reshape(_PK, 128)
    y1p = xwp * dinvp

    p1 = _sc_aggregate(y1p.reshape(_NPAD, _HID), e4, zeros_pad)
    y2p = _tc_layer2(p1.reshape(_NC, _PK, 128), y1p, dinvp, b1p, Wb2)
    p2 = _sc_aggregate(y2p.reshape(_NPAD, _HID), e4, zeros_pad)
    lsp = _tc_layer3(p2.reshape(_NC, _PK, 128), y2p, dinvp, b2p, R)
    return lsp.reshape(_N, _HID)


# speedup vs baseline: 1.1199x; 1.1199x over previous
import functools

import jax
import jax.numpy as jnp
from jax import lax
from jax.experimental import pallas as pl
from jax.experimental.pallas import tpu as pltpu
from jax.experimental.pallas import tpu_sc as plsc

_N = 10000
_E = 320000
_F_IN = 128
_HID = 16

_NC = 2
_NS = 16
_NW = _NC * _NS
_EPW = _E // _NW
_NCH = 80
_CH = 125
_NPAD = 10112
_STRIPE = _NPAD // _NS
_PK = _NPAD // 8
_PKN = _N // 8


def _sc_mesh():
    return plsc.VectorSubcoreMesh(
        core_axis_name="c", subcore_axis_name="s",
        num_cores=_NC, num_subcores=_NS)


def _sc_degree(e5):
    @functools.partial(
        pl.kernel,
        out_type=jax.ShapeDtypeStruct((_NW, _NPAD), jnp.float32),
        mesh=_sc_mesh(),
        scratch_types=[
            pltpu.VMEM((_EPW // 16, 16), jnp.int32),
            pltpu.VMEM((_NPAD,), jnp.float32),
        ],
        compiler_params=pltpu.CompilerParams(
            needs_layout_passes=False, use_tc_tiling_on_sc=False),
    )
    def deg_kernel(e_hbm, deg_hbm, dst_v, deg_v):
        cid = lax.axis_index("c")
        sid = lax.axis_index("s")
        wid = sid * _NC + cid
        pltpu.sync_copy(e_hbm.at[1, wid], dst_v)

        zero16 = jnp.zeros((16,), jnp.float32)

        @pl.loop(0, _NPAD // 16)
        def _zero(i):
            deg_v[pl.ds(i * 16, 16)] = zero16

        one16 = jnp.ones((16,), jnp.float32)

        @pl.loop(0, _EPW // 16)
        def _hist(i):
            idx = dst_v[i]
            plsc.addupdate_scatter(deg_v, [idx], one16)

        pltpu.sync_copy(deg_v, deg_hbm.at[wid])

    return deg_kernel(e5)


def _sc_aggregate(y, e4, zeros_pad):
    nbuf = 8
    pre = 4

    @functools.partial(
        pl.kernel,
        out_type=jax.ShapeDtypeStruct((_NC, _NPAD, _HID), jnp.float32),
        mesh=_sc_mesh(),
        scratch_types=[
            pltpu.VMEM((_NCH, _CH), jnp.int32),
            pltpu.VMEM((_NCH, _CH), jnp.int32),
            [pltpu.VMEM((_CH, _HID), jnp.float32) for _ in range(nbuf)],
            pltpu.VMEM((_STRIPE, _HID), jnp.float32),
            pltpu.VMEM_SHARED((_NPAD, _HID), jnp.float32),
            [pltpu.SemaphoreType.DMA for _ in range(nbuf)],
            [pltpu.SemaphoreType.DMA for _ in range(nbuf)],
        ],
        compiler_params=pltpu.CompilerParams(use_tc_tiling_on_sc=False),
    )
    def agg_kernel(y_hbm, e_hbm, zero_hbm, out_hbm,
                   src_v, dst_v, rows_v, stage_v, acc, gsem, ssem):
        cid = lax.axis_index("c")
        sid = lax.axis_index("s")
        wid = sid * _NC + cid

        pltpu.sync_copy(e_hbm.at[0, wid], src_v)
        pltpu.sync_copy(e_hbm.at[1, wid], dst_v)
        pltpu.sync_copy(zero_hbm.at[pl.ds(sid * _STRIPE, _STRIPE)],
                        acc.at[pl.ds(sid * _STRIPE, _STRIPE)])
        plsc.subcore_barrier()

        for b in range(pre):
            pltpu.async_copy(y_hbm.at[src_v.at[b]], rows_v[b], gsem[b])

        @pl.loop(0, _NCH // nbuf)
        def _blk(jj):
            for b in range(nbuf):
                c = jj * nbuf + b
                pltpu.make_async_copy(y_hbm.at[src_v.at[c]],
                                      rows_v[b], gsem[b]).wait()
                pltpu.async_copy(rows_v[b], acc.at[dst_v.at[c]],
                                 ssem[b], add=True)
                b2 = (b + pre) % nbuf

                @pl.when(c >= nbuf - pre)
                def _drain():
                    pltpu.make_async_copy(rows_v[b2], acc.at[dst_v.at[c]],
                                          ssem[b2]).wait()

                @pl.when(c < _NCH - pre)
                def _prefetch():
                    pltpu.async_copy(y_hbm.at[src_v.at[c + pre]],
                                     rows_v[b2], gsem[b2])

        for k in range(nbuf - pre):
            b2 = (_NCH + pre - nbuf + k) % nbuf
            pltpu.make_async_copy(rows_v[b2], acc.at[dst_v.at[0]],
                                  ssem[b2]).wait()

        plsc.subcore_barrier()
        pltpu.sync_copy(acc.at[pl.ds(sid * _STRIPE, _STRIPE)], stage_v)
        pltpu.sync_copy(stage_v,
                        out_hbm.at[cid, pl.ds(sid * _STRIPE, _STRIPE)])

    return agg_kernel(y, e4, zeros_pad)


def _tc_matmul1(xp, Wb1):
    def body(x_ref, w_ref, xw_ref):
        xw = jnp.dot(x_ref[...], w_ref[...], preferred_element_type=jnp.float32)
        xw_ref[0:_PKN] = xw
        xw_ref[_PKN:_PK] = jnp.zeros((_PK - _PKN, 128), jnp.float32)

    return pl.pallas_call(
        body,
        out_shape=jax.ShapeDtypeStruct((_PK, 128), jnp.float32),
    )(xp, Wb1)


def _tc_dinv(deg_part):
    def body(deg_ref, dinv_ref):
        deg = jnp.sum(deg_ref[...], axis=0, keepdims=True)
        dinv_ref[...] = lax.rsqrt(deg + 1.0)

    return pl.pallas_call(
        body,
        out_shape=jax.ShapeDtypeStruct((1, _NPAD), jnp.float32),
    )(deg_part)


def _tc_layer2(p, y1p, dinvp, b1p, Wb2):
    def body(p_ref, y_ref, d_ref, b_ref, w_ref, out_ref):
        d = d_ref[...]
        h = jnp.maximum(d * (p_ref[0] + p_ref[1] + y_ref[...]) + b_ref[...],
                        0.0)
        out_ref[...] = jnp.dot(
            h, w_ref[...], preferred_element_type=jnp.float32) * d

    return pl.pallas_call(
        body,
        out_shape=jax.ShapeDtypeStruct((_PK, 128), jnp.float32),
    )(p, y1p, dinvp, b1p, Wb2)


def _tc_layer3(p, y2p, dinvp, b2p, R):
    def body(p_ref, y_ref, d_ref, b_ref, r_ref, out_ref):
        o = d_ref[...] * (p_ref[0] + p_ref[1] + y_ref[...]) + b_ref[...]
        s = o - jnp.max(o)
        e = jnp.exp(s)
        r = r_ref[...]
        t = lax.dot_general(e, r, (((1,), (1,)), ((), ())),
                            preferred_element_type=jnp.float32)
        lsp = s - jnp.dot(jnp.log(t), r, preferred_element_type=jnp.float32)
        out_ref[...] = lsp[0:_PKN]

    return pl.pallas_call(
        body,
        out_shape=jax.ShapeDtypeStruct((_PKN, 128), jnp.float32),
    )(p, y2p, dinvp, b2p, R)


def kernel(x, edge_index, W1, b1, W2, b2):
    eye8 = jnp.eye(8, dtype=jnp.float32)
    Wb1 = jnp.kron(eye8, W1)
    Wb2 = jnp.kron(eye8, W2)
    R = jnp.kron(eye8, jnp.ones((1, 16), jnp.float32))
    b1p = jnp.tile(b1, 8).reshape(1, 128)
    b2p = jnp.tile(b2, 8).reshape(1, 128)
    xp = x.reshape(_PKN, 8 * _F_IN)
    e4 = edge_index.reshape(2, _NW, _NCH, _CH)
    e5 = edge_index.reshape(2, _NW, _EPW // 16, 16)
    zeros_pad = jnp.zeros((_NPAD, _HID), jnp.float32)

    deg_part = _sc_degree(e5)
    xwp = _tc_matmul1(xp, Wb1)
    dinv = _tc_dinv(deg_part)
    dinvp = jnp.broadcast_to(
        dinv.reshape(_PK, 8, 1), (_PK, 8, _HID)).reshape(_PK, 128)
    y1p = xwp * dinvp

    p1 = _sc_aggregate(y1p.reshape(_NPAD, _HID), e4, zeros_pad)
    y2p = _tc_layer2(p1.reshape(_NC, _PK, 128), y1p, dinvp, b1p, Wb2)
    p2 = _sc_aggregate(y2p.reshape(_NPAD, _HID), e4, zeros_pad)
    lsp = _tc_layer3(p2.reshape(_NC, _PK, 128), y2p, dinvp, b2p, R)
    return lsp.reshape(_N, _HID)


# speedup vs baseline: 1.2707x; 1.1346x over previous
import functools

import jax
import jax.numpy as jnp
from jax import lax
from jax.experimental import pallas as pl
from jax.experimental.pallas import tpu as pltpu
from jax.experimental.pallas import tpu_sc as plsc

_N = 10000
_E = 320000
_F_IN = 128
_HID = 16

_NC = 2
_NS = 16
_NW = _NC * _NS
_EPW = _E // _NW
_NCH = 80
_CH = 125
_NPAD = 10112
_STRIPE = _NPAD // _NS
_PK = _NPAD // 8
_PKN = _N // 8


def _sc_mesh():
    return plsc.VectorSubcoreMesh(
        core_axis_name="c", subcore_axis_name="s",
        num_cores=_NC, num_subcores=_NS)


def _sc_degree(e4):
    nfull = _CH // 16
    rem = _CH - nfull * 16

    @functools.partial(
        pl.kernel,
        out_type=jax.ShapeDtypeStruct((_NW, _NPAD), jnp.float32),
        mesh=_sc_mesh(),
        scratch_types=[
            pltpu.VMEM((_NCH, _CH), jnp.int32),
            pltpu.VMEM((_NPAD,), jnp.float32),
        ],
        compiler_params=pltpu.CompilerParams(
            needs_layout_passes=False, use_tc_tiling_on_sc=False),
    )
    def deg_kernel(e_hbm, deg_hbm, dst_v, deg_v):
        cid = lax.axis_index("c")
        sid = lax.axis_index("s")
        wid = sid * _NC + cid
        pltpu.sync_copy(e_hbm.at[1, wid], dst_v)

        zero16 = jnp.zeros((16,), jnp.float32)

        @pl.loop(0, _NPAD // 16)
        def _zero(i):
            deg_v[pl.ds(i * 16, 16)] = zero16

        one16 = jnp.ones((16,), jnp.float32)
        tailmask = jnp.arange(16, dtype=jnp.int32) >= (16 - rem)

        @pl.loop(0, _NCH)
        def _hist(j):
            for k in range(nfull):
                idx = dst_v[j, pl.ds(k * 16, 16)]
                plsc.addupdate_scatter(deg_v, [idx], one16)
            idx = dst_v[j, pl.ds(_CH - 16, 16)]
            plsc.addupdate_scatter(deg_v, [idx], one16, mask=tailmask)

        pltpu.sync_copy(deg_v, deg_hbm.at[wid])

    return deg_kernel(e4)


def _sc_aggregate(y, e4, zeros_pad):
    nbuf = 8
    pre = 6

    @functools.partial(
        pl.kernel,
        out_type=jax.ShapeDtypeStruct((_NC, _NPAD, _HID), jnp.float32),
        mesh=_sc_mesh(),
        scratch_types=[
            pltpu.VMEM((_NCH, _CH), jnp.int32),
            pltpu.VMEM((_NCH, _CH), jnp.int32),
            [pltpu.VMEM((_CH, _HID), jnp.float32) for _ in range(nbuf)],
            pltpu.VMEM((_STRIPE, _HID), jnp.float32),
            pltpu.VMEM_SHARED((_NPAD, _HID), jnp.float32),
            [pltpu.SemaphoreType.DMA for _ in range(nbuf)],
            [pltpu.SemaphoreType.DMA for _ in range(nbuf)],
        ],
        compiler_params=pltpu.CompilerParams(use_tc_tiling_on_sc=False),
    )
    def agg_kernel(y_hbm, e_hbm, zero_hbm, out_hbm,
                   src_v, dst_v, rows_v, stage_v, acc, gsem, ssem):
        cid = lax.axis_index("c")
        sid = lax.axis_index("s")
        wid = sid * _NC + cid

        pltpu.sync_copy(e_hbm.at[0, wid], src_v)
        pltpu.sync_copy(e_hbm.at[1, wid], dst_v)
        pltpu.sync_copy(zero_hbm.at[pl.ds(sid * _STRIPE, _STRIPE)],
                        acc.at[pl.ds(sid * _STRIPE, _STRIPE)])
        plsc.subcore_barrier()

        for b in range(pre):
            pltpu.async_copy(y_hbm.at[src_v.at[b]], rows_v[b], gsem[b])

        @pl.loop(0, _NCH // nbuf)
        def _blk(jj):
            for b in range(nbuf):
                c = jj * nbuf + b
                pltpu.make_async_copy(y_hbm.at[src_v.at[c]],
                                      rows_v[b], gsem[b]).wait()
                pltpu.async_copy(rows_v[b], acc.at[dst_v.at[c]],
                                 ssem[b], add=True)
                b2 = (b + pre) % nbuf

                @pl.when(c >= nbuf - pre)
                def _drain():
                    pltpu.make_async_copy(rows_v[b2], acc.at[dst_v.at[c]],
                                          ssem[b2]).wait()

                @pl.when(c < _NCH - pre)
                def _prefetch():
                    pltpu.async_copy(y_hbm.at[src_v.at[c + pre]],
                                     rows_v[b2], gsem[b2])

        for k in range(nbuf - pre):
            b2 = (_NCH + pre - nbuf + k) % nbuf
            pltpu.make_async_copy(rows_v[b2], acc.at[dst_v.at[0]],
                                  ssem[b2]).wait()

        plsc.subcore_barrier()
        pltpu.sync_copy(acc.at[pl.ds(sid * _STRIPE, _STRIPE)], stage_v)
        pltpu.sync_copy(stage_v,
                        out_hbm.at[cid, pl.ds(sid * _STRIPE, _STRIPE)])

    return agg_kernel(y, e4, zeros_pad)


def _tc_matmul1(xp, Wb1):
    def body(x_ref, w_ref, xw_ref):
        xw = jnp.dot(x_ref[...], w_ref[...], preferred_element_type=jnp.float32)
        xw_ref[0:_PKN] = xw
        xw_ref[_PKN:_PK] = jnp.zeros((_PK - _PKN, 128), jnp.float32)

    return pl.pallas_call(
        body,
        out_shape=jax.ShapeDtypeStruct((_PK, 128), jnp.float32),
    )(xp, Wb1)


def _tc_dinv(deg_part):
    def body(deg_ref, dinv_ref):
        deg = jnp.sum(deg_ref[...], axis=0, keepdims=True)
        dinv_ref[...] = lax.rsqrt(deg + 1.0)

    return pl.pallas_call(
        body,
        out_shape=jax.ShapeDtypeStruct((1, _NPAD), jnp.float32),
    )(deg_part)


def _tc_layer2(p, y1p, dinvp, b1p, Wb2):
    def body(p_ref, y_ref, d_ref, b_ref, w_ref, out_ref):
        d = d_ref[...]
        h = jnp.maximum(d * (p_ref[0] + p_ref[1] + y_ref[...]) + b_ref[...],
                        0.0)
        out_ref[...] = jnp.dot(
            h, w_ref[...], preferred_element_type=jnp.float32) * d

    return pl.pallas_call(
        body,
        out_shape=jax.ShapeDtypeStruct((_PK, 128), jnp.float32),
    )(p, y1p, dinvp, b1p, Wb2)


def _tc_layer3(p, y2p, dinvp, b2p, R):
    def body(p_ref, y_ref, d_ref, b_ref, r_ref, out_ref):
        o = d_ref[...] * (p_ref[0] + p_ref[1] + y_ref[...]) + b_ref[...]
        s = o - jnp.max(o)
        e = jnp.exp(s)
        r = r_ref[...]
        t = lax.dot_general(e, r, (((1,), (1,)), ((), ())),
                            preferred_element_type=jnp.float32)
        lsp = s - jnp.dot(jnp.log(t), r, preferred_element_type=jnp.float32)
        out_ref[...] = lsp[0:_PKN]

    return pl.pallas_call(
        body,
        out_shape=jax.ShapeDtypeStruct((_PKN, 128), jnp.float32),
    )(p, y2p, dinvp, b2p, R)


def kernel(x, edge_index, W1, b1, W2, b2):
    eye8 = jnp.eye(8, dtype=jnp.float32)
    Wb1 = jnp.kron(eye8, W1)
    Wb2 = jnp.kron(eye8, W2)
    R = jnp.kron(eye8, jnp.ones((1, 16), jnp.float32))
    b1p = jnp.tile(b1, 8).reshape(1, 128)
    b2p = jnp.tile(b2, 8).reshape(1, 128)
    xp = x.reshape(_PKN, 8 * _F_IN)
    e4 = edge_index.reshape(2, _NW, _NCH, _CH)
    zeros_pad = jnp.zeros((_NPAD, _HID), jnp.float32)

    deg_part = _sc_degree(e4)
    xwp = _tc_matmul1(xp, Wb1)
    dinv = _tc_dinv(deg_part)
    dinvp = jnp.broadcast_to(
        dinv.reshape(_PK, 8, 1), (_PK, 8, _HID)).reshape(_PK, 128)
    y1p = xwp * dinvp

    p1 = _sc_aggregate(y1p.reshape(_NPAD, _HID), e4, zeros_pad)
    y2p = _tc_layer2(p1.reshape(_NC, _PK, 128), y1p, dinvp, b1p, Wb2)
    p2 = _sc_aggregate(y2p.reshape(_NPAD, _HID), e4, zeros_pad)
    lsp = _tc_layer3(p2.reshape(_NC, _PK, 128), y2p, dinvp, b2p, R)
    return lsp.reshape(_N, _HID)


# speedup vs baseline: 1.2928x; 1.0174x over previous
import functools

import jax
import jax.numpy as jnp
from jax import lax
from jax.experimental import pallas as pl
from jax.experimental.pallas import tpu as pltpu
from jax.experimental.pallas import tpu_sc as plsc

_N = 10000
_E = 320000
_F_IN = 128
_HID = 16

_NC = 2
_NS = 16
_NW = _NC * _NS
_EPW = _E // _NW
_NCH = 80
_CH = 125
_NPAD = 10112
_STRIPE = _NPAD // _NS
_PK = _NPAD // 8
_PKN = _N // 8


def _sc_mesh():
    return plsc.VectorSubcoreMesh(
        core_axis_name="c", subcore_axis_name="s",
        num_cores=_NC, num_subcores=_NS)


def _sc_degree(e4):
    nfull = _CH // 16
    rem = _CH - nfull * 16

    @functools.partial(
        pl.kernel,
        out_type=jax.ShapeDtypeStruct((_NW, _NPAD), jnp.float32),
        mesh=_sc_mesh(),
        scratch_types=[
            pltpu.VMEM((_NCH, _CH), jnp.int32),
            pltpu.VMEM((_NPAD,), jnp.float32),
        ],
        compiler_params=pltpu.CompilerParams(
            needs_layout_passes=False, use_tc_tiling_on_sc=False),
    )
    def deg_kernel(e_hbm, deg_hbm, dst_v, deg_v):
        cid = lax.axis_index("c")
        sid = lax.axis_index("s")
        wid = sid * _NC + cid
        pltpu.sync_copy(e_hbm.at[1, wid], dst_v)

        zero16 = jnp.zeros((16,), jnp.float32)

        @pl.loop(0, _NPAD // 16)
        def _zero(i):
            deg_v[pl.ds(i * 16, 16)] = zero16

        one16 = jnp.ones((16,), jnp.float32)
        tailmask = jnp.arange(16, dtype=jnp.int32) >= (16 - rem)

        @pl.loop(0, _NCH)
        def _hist(j):
            for k in range(nfull):
                idx = dst_v[j, pl.ds(k * 16, 16)]
                plsc.addupdate_scatter(deg_v, [idx], one16)
            idx = dst_v[j, pl.ds(_CH - 16, 16)]
            plsc.addupdate_scatter(deg_v, [idx], one16, mask=tailmask)

        pltpu.sync_copy(deg_v, deg_hbm.at[wid])

    return deg_kernel(e4)


def _sc_aggregate(y, e4):
    nbuf = 8
    pre = 6

    @functools.partial(
        pl.kernel,
        out_type=jax.ShapeDtypeStruct((_NC, _NPAD, _HID), jnp.float32),
        mesh=_sc_mesh(),
        scratch_types=[
            pltpu.VMEM((_NCH, _CH), jnp.int32),
            pltpu.VMEM((_NCH, _CH), jnp.int32),
            [pltpu.VMEM((_CH, _HID), jnp.float32) for _ in range(nbuf)],
            pltpu.VMEM((_STRIPE, _HID), jnp.float32),
            pltpu.VMEM_SHARED((_NPAD, _HID), jnp.float32),
            [pltpu.SemaphoreType.DMA for _ in range(nbuf)],
            [pltpu.SemaphoreType.DMA for _ in range(nbuf)],
        ],
        compiler_params=pltpu.CompilerParams(use_tc_tiling_on_sc=False),
    )
    def agg_kernel(y_hbm, e_hbm, out_hbm,
                   src_v, dst_v, rows_v, stage_v, acc, gsem, ssem):
        cid = lax.axis_index("c")
        sid = lax.axis_index("s")
        wid = sid * _NC + cid

        pltpu.sync_copy(e_hbm.at[0, wid], src_v)
        pltpu.sync_copy(e_hbm.at[1, wid], dst_v)
        zero16 = jnp.zeros((16,), jnp.float32)

        @pl.loop(0, _STRIPE)
        def _zero(i):
            stage_v[i] = zero16

        pltpu.sync_copy(stage_v, acc.at[pl.ds(sid * _STRIPE, _STRIPE)])
        plsc.subcore_barrier()

        for b in range(pre):
            pltpu.async_copy(y_hbm.at[src_v.at[b]], rows_v[b], gsem[b])

        @pl.loop(0, _NCH // nbuf)
        def _blk(jj):
            for b in range(nbuf):
                c = jj * nbuf + b
                pltpu.make_async_copy(y_hbm.at[src_v.at[c]],
                                      rows_v[b], gsem[b]).wait()
                pltpu.async_copy(rows_v[b], acc.at[dst_v.at[c]],
                                 ssem[b], add=True)
                b2 = (b + pre) % nbuf

                @pl.when(c >= nbuf - pre)
                def _drain():
                    pltpu.make_async_copy(rows_v[b2], acc.at[dst_v.at[c]],
                                          ssem[b2]).wait()

                @pl.when(c < _NCH - pre)
                def _prefetch():
                    pltpu.async_copy(y_hbm.at[src_v.at[c + pre]],
                                     rows_v[b2], gsem[b2])

        for k in range(nbuf - pre):
            b2 = (_NCH + pre - nbuf + k) % nbuf
            pltpu.make_async_copy(rows_v[b2], acc.at[dst_v.at[0]],
                                  ssem[b2]).wait()

        plsc.subcore_barrier()
        pltpu.sync_copy(acc.at[pl.ds(sid * _STRIPE, _STRIPE)], stage_v)
        pltpu.sync_copy(stage_v,
                        out_hbm.at[cid, pl.ds(sid * _STRIPE, _STRIPE)])

    return agg_kernel(y, e4)


def _tc_matmul1(xp, Wb1):
    def body(x_ref, w_ref, xw_ref):
        xw = jnp.dot(x_ref[...], w_ref[...], preferred_element_type=jnp.float32)
        xw_ref[0:_PKN] = xw
        xw_ref[_PKN:_PK] = jnp.zeros((_PK - _PKN, 128), jnp.float32)

    return pl.pallas_call(
        body,
        out_shape=jax.ShapeDtypeStruct((_PK, 128), jnp.float32),
    )(xp, Wb1)


def _tc_dinv(deg_flat):
    rows = _NPAD // 128

    def body(deg_ref, dinv_ref):
        deg = deg_ref[0:rows, :]
        for w in range(1, _NW):
            deg = deg + deg_ref[w * rows:(w + 1) * rows, :]
        dinv_ref[0:rows] = lax.rsqrt(deg + 1.0)
        dinv_ref[rows:rows + 1] = jnp.ones((1, 128), jnp.float32)

    return pl.pallas_call(
        body,
        out_shape=jax.ShapeDtypeStruct((rows + 1, 128), jnp.float32),
    )(deg_flat)


def _tc_layer2(p, y1p, dinvp, b1p, Wb2):
    def body(p_ref, y_ref, d_ref, b_ref, w_ref, out_ref):
        d = d_ref[...]
        h = jnp.maximum(d * (p_ref[0] + p_ref[1] + y_ref[...]) + b_ref[...],
                        0.0)
        out_ref[...] = jnp.dot(
            h, w_ref[...], preferred_element_type=jnp.float32) * d

    return pl.pallas_call(
        body,
        out_shape=jax.ShapeDtypeStruct((_PK, 128), jnp.float32),
    )(p, y1p, dinvp, b1p, Wb2)


def _tc_layer3(p, y2p, dinvp, b2p, R):
    def body(p_ref, y_ref, d_ref, b_ref, r_ref, out_ref):
        o = d_ref[...] * (p_ref[0] + p_ref[1] + y_ref[...]) + b_ref[...]
        s = o - jnp.max(o)
        e = jnp.exp(s)
        r = r_ref[...]
        t = lax.dot_general(e, r, (((1,), (1,)), ((), ())),
                            preferred_element_type=jnp.float32)
        lsp = s - jnp.dot(jnp.log(t), r, preferred_element_type=jnp.float32)
        out_ref[...] = lsp[0:_PKN]

    return pl.pallas_call(
        body,
        out_shape=jax.ShapeDtypeStruct((_PKN, 128), jnp.float32),
    )(p, y2p, dinvp, b2p, R)


def kernel(x, edge_index, W1, b1, W2, b2):
    eye8 = jnp.eye(8, dtype=jnp.float32)
    Wb1 = jnp.kron(eye8, W1)
    Wb2 = jnp.kron(eye8, W2)
    R = jnp.kron(eye8, jnp.ones((1, 16), jnp.float32))
    b1p = jnp.tile(b1, 8).reshape(1, 128)
    b2p = jnp.tile(b2, 8).reshape(1, 128)
    xp = x.reshape(_PKN, 8 * _F_IN)
    e4 = edge_index.reshape(2, _NW, _NCH, _CH)

    deg_part = _sc_degree(e4)
    xwp = _tc_matmul1(xp, Wb1)
    dinv = _tc_dinv(deg_part.reshape(_NW * (_NPAD // 128), 128))
    dinvp = jnp.broadcast_to(
        dinv.reshape(-1)[:_NPAD].reshape(_PK, 8, 1),
        (_PK, 8, _HID)).reshape(_PK, 128)
    y1p = xwp * dinvp

    p1 = _sc_aggregate(y1p.reshape(_NPAD, _HID), e4)
    y2p = _tc_layer2(p1.reshape(_NC, _PK, 128), y1p, dinvp, b1p, Wb2)
    p2 = _sc_aggregate(y2p.reshape(_NPAD, _HID), e4)
    lsp = _tc_layer3(p2.reshape(_NC, _PK, 128), y2p, dinvp, b2p, R)
    return lsp.reshape(_N, _HID)


# speedup vs baseline: 1.3036x; 1.0084x over previous
import functools

import jax
import jax.numpy as jnp
from jax import lax
from jax.experimental import pallas as pl
from jax.experimental.pallas import tpu as pltpu
from jax.experimental.pallas import tpu_sc as plsc

_N = 10000
_E = 320000
_F_IN = 128
_HID = 16

_NC = 2
_NS = 16
_NW = _NC * _NS
_EPW = _E // _NW
_NCH = 80
_CH = 125
_NPAD = 10112
_STRIPE = _NPAD // _NS
_PK = _NPAD // 8
_PKN = _N // 8


def _sc_mesh():
    return plsc.VectorSubcoreMesh(
        core_axis_name="c", subcore_axis_name="s",
        num_cores=_NC, num_subcores=_NS)


def _sc_degree(e4):
    nfull = _CH // 16
    rem = _CH - nfull * 16

    @functools.partial(
        pl.kernel,
        out_type=jax.ShapeDtypeStruct((_NW, _NPAD), jnp.float32),
        mesh=_sc_mesh(),
        scratch_types=[
            pltpu.VMEM((_NCH, _CH), jnp.int32),
            pltpu.VMEM((_NPAD,), jnp.float32),
        ],
        compiler_params=pltpu.CompilerParams(
            needs_layout_passes=False, use_tc_tiling_on_sc=False),
    )
    def deg_kernel(e_hbm, deg_hbm, dst_v, deg_v):
        cid = lax.axis_index("c")
        sid = lax.axis_index("s")
        wid = sid * _NC + cid
        pltpu.sync_copy(e_hbm.at[1, wid], dst_v)

        zero16 = jnp.zeros((16,), jnp.float32)

        @pl.loop(0, _NPAD // 16)
        def _zero(i):
            deg_v[pl.ds(i * 16, 16)] = zero16

        one16 = jnp.ones((16,), jnp.float32)
        tailmask = jnp.arange(16, dtype=jnp.int32) >= (16 - rem)

        @pl.loop(0, _NCH)
        def _hist(j):
            for k in range(nfull):
                idx = dst_v[j, pl.ds(k * 16, 16)]
                plsc.addupdate_scatter(deg_v, [idx], one16)
            idx = dst_v[j, pl.ds(_CH - 16, 16)]
            plsc.addupdate_scatter(deg_v, [idx], one16, mask=tailmask)

        pltpu.sync_copy(deg_v, deg_hbm.at[wid])

    return deg_kernel(e4)


def _sc_aggregate(y, e4):
    nbuf = 10
    pre = 7

    @functools.partial(
        pl.kernel,
        out_type=jax.ShapeDtypeStruct((_NC, _NPAD, _HID), jnp.float32),
        mesh=_sc_mesh(),
        scratch_types=[
            pltpu.VMEM((_NCH, _CH), jnp.int32),
            pltpu.VMEM((_NCH, _CH), jnp.int32),
            [pltpu.VMEM((_CH, _HID), jnp.float32) for _ in range(nbuf)],
            pltpu.VMEM((_STRIPE, _HID), jnp.float32),
            pltpu.VMEM_SHARED((_NPAD, _HID), jnp.float32),
            [pltpu.SemaphoreType.DMA for _ in range(nbuf)],
            [pltpu.SemaphoreType.DMA for _ in range(nbuf)],
        ],
        compiler_params=pltpu.CompilerParams(use_tc_tiling_on_sc=False),
    )
    def agg_kernel(y_hbm, e_hbm, out_hbm,
                   src_v, dst_v, rows_v, stage_v, acc, gsem, ssem):
        cid = lax.axis_index("c")
        sid = lax.axis_index("s")
        wid = sid * _NC + cid

        pltpu.sync_copy(e_hbm.at[0, wid], src_v)
        pltpu.sync_copy(e_hbm.at[1, wid], dst_v)
        zero16 = jnp.zeros((16,), jnp.float32)

        @pl.loop(0, _STRIPE)
        def _zero(i):
            stage_v[i] = zero16

        pltpu.sync_copy(stage_v, acc.at[pl.ds(sid * _STRIPE, _STRIPE)])
        plsc.subcore_barrier()

        for b in range(pre):
            pltpu.async_copy(y_hbm.at[src_v.at[b]], rows_v[b], gsem[b])

        @pl.loop(0, _NCH // nbuf)
        def _blk(jj):
            for b in range(nbuf):
                c = jj * nbuf + b
                pltpu.make_async_copy(y_hbm.at[src_v.at[c]],
                                      rows_v[b], gsem[b]).wait()
                pltpu.async_copy(rows_v[b], acc.at[dst_v.at[c]],
                                 ssem[b], add=True)
                b2 = (b + pre) % nbuf

                @pl.when(c >= nbuf - pre)
                def _drain():
                    pltpu.make_async_copy(rows_v[b2], acc.at[dst_v.at[c]],
                                          ssem[b2]).wait()

                @pl.when(c < _NCH - pre)
                def _prefetch():
                    pltpu.async_copy(y_hbm.at[src_v.at[c + pre]],
                                     rows_v[b2], gsem[b2])

        for k in range(nbuf - pre):
            b2 = (_NCH + pre - nbuf + k) % nbuf
            pltpu.make_async_copy(rows_v[b2], acc.at[dst_v.at[0]],
                                  ssem[b2]).wait()

        plsc.subcore_barrier()
        pltpu.sync_copy(acc.at[pl.ds(sid * _STRIPE, _STRIPE)], stage_v)
        pltpu.sync_copy(stage_v,
                        out_hbm.at[cid, pl.ds(sid * _STRIPE, _STRIPE)])

    return agg_kernel(y, e4)


def _tc_matmul1(xp, Wb1):
    def body(x_ref, w_ref, xw_ref):
        xw = jnp.dot(x_ref[...], w_ref[...], preferred_element_type=jnp.float32)
        xw_ref[0:_PKN] = xw
        xw_ref[_PKN:_PK] = jnp.zeros((_PK - _PKN, 128), jnp.float32)

    return pl.pallas_call(
        body,
        out_shape=jax.ShapeDtypeStruct((_PK, 128), jnp.float32),
    )(xp, Wb1)


def _tc_dinv(deg_flat):
    rows = _NPAD // 128

    def body(deg_ref, dinv_ref):
        deg = deg_ref[0:rows, :]
        for w in range(1, _NW):
            deg = deg + deg_ref[w * rows:(w + 1) * rows, :]
        dinv_ref[0:rows] = lax.rsqrt(deg + 1.0)
        dinv_ref[rows:rows + 1] = jnp.ones((1, 128), jnp.float32)

    return pl.pallas_call(
        body,
        out_shape=jax.ShapeDtypeStruct((rows + 1, 128), jnp.float32),
    )(deg_flat)


def _tc_layer2(p, y1p, dinvp, b1p, Wb2):
    def body(p_ref, y_ref, d_ref, b_ref, w_ref, out_ref):
        d = d_ref[...]
        h = jnp.maximum(d * (p_ref[0] + p_ref[1] + y_ref[...]) + b_ref[...],
                        0.0)
        out_ref[...] = jnp.dot(
            h, w_ref[...], preferred_element_type=jnp.float32) * d

    return pl.pallas_call(
        body,
        out_shape=jax.ShapeDtypeStruct((_PK, 128), jnp.float32),
    )(p, y1p, dinvp, b1p, Wb2)


def _tc_layer3(p, y2p, dinvp, b2p, R):
    def body(p_ref, y_ref, d_ref, b_ref, r_ref, out_ref):
        o = d_ref[...] * (p_ref[0] + p_ref[1] + y_ref[...]) + b_ref[...]
        s = o - jnp.max(o)
        e = jnp.exp(s)
        r = r_ref[...]
        t = lax.dot_general(e, r, (((1,), (1,)), ((), ())),
                            preferred_element_type=jnp.float32)
        lsp = s - jnp.dot(jnp.log(t), r, preferred_element_type=jnp.float32)
        out_ref[...] = lsp[0:_PKN]

    return pl.pallas_call(
        body,
        out_shape=jax.ShapeDtypeStruct((_PKN, 128), jnp.float32),
    )(p, y2p, dinvp, b2p, R)


def kernel(x, edge_index, W1, b1, W2, b2):
    eye8 = jnp.eye(8, dtype=jnp.float32)
    Wb1 = jnp.kron(eye8, W1)
    Wb2 = jnp.kron(eye8, W2)
    R = jnp.kron(eye8, jnp.ones((1, 16), jnp.float32))
    b1p = jnp.tile(b1, 8).reshape(1, 128)
    b2p = jnp.tile(b2, 8).reshape(1, 128)
    xp = x.reshape(_PKN, 8 * _F_IN)
    e4 = edge_index.reshape(2, _NW, _NCH, _CH)

    deg_part = _sc_degree(e4)
    xwp = _tc_matmul1(xp, Wb1)
    dinv = _tc_dinv(deg_part.reshape(_NW * (_NPAD // 128), 128))
    dinvp = jnp.broadcast_to(
        dinv.reshape(-1)[:_NPAD].reshape(_PK, 8, 1),
        (_PK, 8, _HID)).reshape(_PK, 128)
    y1p = xwp * dinvp

    p1 = _sc_aggregate(y1p.reshape(_NPAD, _HID), e4)
    y2p = _tc_layer2(p1.reshape(_NC, _PK, 128), y1p, dinvp, b1p, Wb2)
    p2 = _sc_aggregate(y2p.reshape(_NPAD, _HID), e4)
    lsp = _tc_layer3(p2.reshape(_NC, _PK, 128), y2p, dinvp, b2p, R)
    return lsp.reshape(_N, _HID)
